# baseline (device time: 56070 ns/iter reference)
import jax
import jax.numpy as jnp
from jax import lax
from jax.experimental import pallas as pl
from jax.experimental.pallas import tpu as pltpu

N_DEV = 8
N_HOP = N_DEV - 1
M = 1024
M_HALF = M // 2
N_CHUNK = 512
K_SUB = 4
N_SUB = N_CHUNK // K_SUB


def kernel(x):
    x = x.reshape(M, N_DEV * N_CHUNK)

    def body(
        x_ref,
        out_ref,
        xv,
        comm_a,
        comm_b,
        acc_a,
        acc_b,
        copy_sems,
        send_sems_a,
        send_sems_b,
        recv_sems_a,
        recv_sems_b,
    ):
        my = lax.axis_index("i")
        left = lax.rem(my + N_DEV - 1, N_DEV)
        right = lax.rem(my + 1, N_DEV)

        def xcopy(j):
            c = lax.rem(my + j, N_DEV)
            return pltpu.make_async_copy(
                x_ref.at[:, pl.ds(c * N_CHUNK, N_CHUNK)],
                xv.at[j],
                copy_sems.at[j],
            )

        stage_order = (7, 1, 6, 2, 5, 3, 4, 0)
        for j in stage_order:
            xcopy(j).start()

        barrier_sem = pltpu.get_barrier_semaphore()
        for nbr in (left, right):
            pl.semaphore_signal(
                barrier_sem, inc=1,
                device_id=(nbr,), device_id_type=pl.DeviceIdType.MESH,
            )
        pl.semaphore_wait(barrier_sem, 2)

        def ksl(k):
            return pl.ds(k * N_SUB, N_SUB)

        def send(dir_tag, h, k):
            comm, ssems, rsems, tgt = (
                (comm_a, send_sems_a, recv_sems_a, right)
                if dir_tag == 0
                else (comm_b, send_sems_b, recv_sems_b, left)
            )
            acc = acc_a if dir_tag == 0 else acc_b
            return pltpu.make_async_remote_copy(
                src_ref=acc.at[k],
                dst_ref=comm.at[h, k],
                send_sem=ssems.at[k],
                recv_sem=rsems.at[h, k],
                device_id=(tgt,),
                device_id_type=pl.DeviceIdType.MESH,
            )

        xcopy(7).wait()
        for k in range(K_SUB):
            acc_a[k] = xv[7, :M_HALF, ksl(k)].astype(jnp.bfloat16)
            send(0, 0, k).start()
        xcopy(1).wait()
        for k in range(K_SUB):
            acc_b[k] = xv[1, M_HALF:, ksl(k)].astype(jnp.bfloat16)
            send(1, 0, k).start()

        waited = {7, 1}
        for h in range(N_HOP):
            ja = 6 - h
            jb = (2 + h) % N_DEV
            for j in (ja, jb):
                if j not in waited:
                    xcopy(j).wait()
                    waited.add(j)
            for k in range(K_SUB):
                send(0, h, k).wait_recv()
                if h < N_HOP - 1:
                    send(0, h, k).wait_send()
                    acc_a[k] = comm_a[h, k] + xv[ja, :M_HALF, ksl(k)].astype(
                        jnp.bfloat16
                    )
                    send(0, h + 1, k).start()
                else:
                    out_ref[:M_HALF, ksl(k)] = comm_a[h, k] + xv[
                        ja, :M_HALF, ksl(k)
                    ].astype(jnp.bfloat16)
                send(1, h, k).wait_recv()
                if h < N_HOP - 1:
                    send(1, h, k).wait_send()
                    acc_b[k] = comm_b[h, k] + xv[jb, M_HALF:, ksl(k)].astype(
                        jnp.bfloat16
                    )
                    send(1, h + 1, k).start()
                else:
                    out_ref[M_HALF:, ksl(k)] = comm_b[h, k] + xv[
                        jb, M_HALF:, ksl(k)
                    ].astype(jnp.bfloat16)

        for k in range(K_SUB):
            send(0, N_HOP - 1, k).wait_send()
            send(1, N_HOP - 1, k).wait_send()

    return pl.pallas_call(
        body,
        out_shape=jax.ShapeDtypeStruct((M, N_CHUNK), jnp.bfloat16),
        in_specs=[pl.BlockSpec(memory_space=pltpu.MemorySpace.HBM)],
        out_specs=pl.BlockSpec(memory_space=pltpu.VMEM),
        scratch_shapes=[
            pltpu.VMEM((N_DEV, M, N_CHUNK), jnp.float32),
            pltpu.VMEM((N_HOP, K_SUB, M_HALF, N_SUB), jnp.bfloat16),
            pltpu.VMEM((N_HOP, K_SUB, M_HALF, N_SUB), jnp.bfloat16),
            pltpu.VMEM((K_SUB, M_HALF, N_SUB), jnp.bfloat16),
            pltpu.VMEM((K_SUB, M_HALF, N_SUB), jnp.bfloat16),
            pltpu.SemaphoreType.DMA((N_DEV,)),
            pltpu.SemaphoreType.DMA((K_SUB,)),
            pltpu.SemaphoreType.DMA((K_SUB,)),
            pltpu.SemaphoreType.DMA((N_HOP, K_SUB)),
            pltpu.SemaphoreType.DMA((N_HOP, K_SUB)),
        ],
        compiler_params=pltpu.CompilerParams(collective_id=0),
    )(x)


# device time: 52842 ns/iter; 1.0611x vs baseline; 1.0611x over previous
import os

import jax
import jax.numpy as jnp
from jax import lax
from jax.experimental import pallas as pl
from jax.experimental.pallas import tpu as pltpu

N_DEV = 8
N_HOP = N_DEV - 1
M = 1024
M_HALF = M // 2
N_CHUNK = 512
K_SUB = int(os.environ.get("RS_K_SUB", "4"))
N_SUB = N_CHUNK // K_SUB
COMM_ONLY = os.environ.get("RS_COMM_ONLY", "0") == "1"


def kernel(x):
    x = x.reshape(M, N_DEV * N_CHUNK)

    def body(
        x_ref,
        out_ref,
        comm_a,
        comm_b,
        acc_a,
        acc_b,
        send_sems_a,
        send_sems_b,
        recv_sems_a,
        recv_sems_b,
    ):
        my = lax.axis_index("i")
        left = lax.rem(my + N_DEV - 1, N_DEV)
        right = lax.rem(my + 1, N_DEV)

        barrier_sem = pltpu.get_barrier_semaphore()
        for nbr in (left, right):
            pl.semaphore_signal(
                barrier_sem, inc=1,
                device_id=(nbr,), device_id_type=pl.DeviceIdType.MESH,
            )
        pl.semaphore_wait(barrier_sem, 2)

        def col(c, k):
            return pl.ds(c * N_CHUNK + k * N_SUB, N_SUB)

        def send(dir_tag, h, k):
            acc, comm, ssems, rsems, tgt = (
                (acc_a, comm_a, send_sems_a, recv_sems_a, right)
                if dir_tag == 0
                else (acc_b, comm_b, send_sems_b, recv_sems_b, left)
            )
            return pltpu.make_async_remote_copy(
                src_ref=acc.at[k],
                dst_ref=comm.at[h, k],
                send_sem=ssems.at[k],
                recv_sem=rsems.at[h, k],
                device_id=(tgt,),
                device_id_type=pl.DeviceIdType.MESH,
            )

        def accum(dir_tag, h, k, dst):
            comm = comm_a if dir_tag == 0 else comm_b
            if COMM_ONLY:
                dst[...] = comm[h, k]
                return
            rows = slice(0, M_HALF) if dir_tag == 0 else slice(M_HALF, M)
            c = (
                lax.rem(my + 2 * N_DEV - 2 - h, N_DEV)
                if dir_tag == 0
                else lax.rem(my + 2 + h, N_DEV)
            )
            dst[...] = comm[h, k] + x_ref[rows, col(c, k)].astype(jnp.bfloat16)

        ca0 = lax.rem(my + N_DEV - 1, N_DEV)
        cb0 = lax.rem(my + 1, N_DEV)
        for k in range(K_SUB):
            acc_a[k] = x_ref[:M_HALF, col(ca0, k)].astype(jnp.bfloat16)
            send(0, 0, k).start()
            acc_b[k] = x_ref[M_HALF:, col(cb0, k)].astype(jnp.bfloat16)
            send(1, 0, k).start()

        for h in range(N_HOP):
            for k in range(K_SUB):
                send(0, h, k).wait_recv()
                if h < N_HOP - 1:
                    send(0, h, k).wait_send()
                    accum(0, h, k, acc_a.at[k])
                    send(0, h + 1, k).start()
                else:
                    accum(0, h, k, out_ref.at[:M_HALF, pl.ds(k * N_SUB, N_SUB)])
                send(1, h, k).wait_recv()
                if h < N_HOP - 1:
                    send(1, h, k).wait_send()
                    accum(1, h, k, acc_b.at[k])
                    send(1, h + 1, k).start()
                else:
                    accum(1, h, k, out_ref.at[M_HALF:, pl.ds(k * N_SUB, N_SUB)])

        for k in range(K_SUB):
            send(0, N_HOP - 1, k).wait_send()
            send(1, N_HOP - 1, k).wait_send()

    return pl.pallas_call(
        body,
        out_shape=jax.ShapeDtypeStruct((M, N_CHUNK), jnp.bfloat16),
        in_specs=[pl.BlockSpec(memory_space=pltpu.VMEM)],
        out_specs=pl.BlockSpec(memory_space=pltpu.VMEM),
        scratch_shapes=[
            pltpu.VMEM((N_HOP, K_SUB, M_HALF, N_SUB), jnp.bfloat16),
            pltpu.VMEM((N_HOP, K_SUB, M_HALF, N_SUB), jnp.bfloat16),
            pltpu.VMEM((K_SUB, M_HALF, N_SUB), jnp.bfloat16),
            pltpu.VMEM((K_SUB, M_HALF, N_SUB), jnp.bfloat16),
            pltpu.SemaphoreType.DMA((K_SUB,)),
            pltpu.SemaphoreType.DMA((K_SUB,)),
            pltpu.SemaphoreType.DMA((N_HOP, K_SUB)),
            pltpu.SemaphoreType.DMA((N_HOP, K_SUB)),
        ],
        compiler_params=pltpu.CompilerParams(collective_id=0),
    )(x)
